# Initial kernel scaffold; baseline (speedup 1.0000x reference)
import jax
import jax.numpy as jnp
from jax.experimental import pallas as pl


def kernel(x, router_logits, w1, w2):
    raise NotImplementedError("write your pallas kernel here")


# speedup vs baseline: 1.6653x; 1.6653x over previous
import jax
import jax.numpy as jnp
from jax import lax
from jax.experimental import pallas as pl
from jax.experimental.pallas import tpu as pltpu
from jax.experimental.pallas import tpu_sc as plsc

E = 8
TOPK = 2
D = 1024
DF = 1024
M = 2048
BM = 256
NB = 24
TPAD = NB * BM
NEG = -1e30


def _route_body(logits_ref, pos_ref, wts_ref, meta_ref):
    lg = logits_ref[...]
    ids = lax.broadcasted_iota(jnp.int32, (M, E), 1)
    mx = jnp.max(lg, axis=1, keepdims=True)
    a1 = jnp.min(jnp.where(lg == mx, ids, E), axis=1, keepdims=True)
    oh0 = ids == a1
    lg2 = jnp.where(oh0, NEG, lg)
    mx2 = jnp.max(lg2, axis=1, keepdims=True)
    a2 = jnp.min(jnp.where(lg2 == mx2, ids, E), axis=1, keepdims=True)
    oh1 = ids == a2
    e2 = jnp.exp(mx2 - mx)
    s = 1.0 + e2
    wts_ref[:, 0:1] = 1.0 / s
    wts_ref[:, 1:2] = e2 / s

    r = (oh0 | oh1).astype(jnp.float32)
    sft = 1
    while sft < M:
        r = r + jnp.concatenate(
            [jnp.zeros((sft, E), jnp.float32), r[: M - sft]], axis=0)
        sft *= 2
    ci = r[M - 1 : M, :].astype(jnp.int32)
    pc = ((ci + (BM - 1)) // BM) * BM
    inc = pc
    sft = 1
    while sft < E:
        inc = inc + jnp.concatenate(
            [jnp.zeros((1, sft), jnp.int32), inc[:, : E - sft]], axis=1)
        sft *= 2
    pstart = inc - pc
    base = pstart + r.astype(jnp.int32) - 1
    pos_ref[:, 0:1] = jnp.sum(jnp.where(oh0, base, 0), axis=1, keepdims=True)
    pos_ref[:, 1:2] = jnp.sum(jnp.where(oh1, base, 0), axis=1, keepdims=True)

    bidx = lax.broadcasted_iota(jnp.int32, (NB, E), 0) * BM
    ends_pad = pstart + pc
    bg = jnp.sum((ends_pad <= bidx).astype(jnp.int32), axis=1, keepdims=True)
    meta_ref[:, 0:1] = jnp.minimum(bg, E - 1)
    real_end = pstart + ci
    meta_ref[:, 1:2] = jnp.sum(
        ((pstart <= bidx) & (bidx < real_end)).astype(jnp.int32),
        axis=1, keepdims=True)


def _route(router_logits):
    return pl.pallas_call(
        _route_body,
        out_shape=[
            jax.ShapeDtypeStruct((M, TOPK), jnp.int32),
            jax.ShapeDtypeStruct((M, TOPK), jnp.float32),
            jax.ShapeDtypeStruct((NB, 2), jnp.int32),
        ],
    )(router_logits)


def _gmm_body(meta_ref, xs_ref, w1_ref, w2_ref, ys_ref):
    i = pl.program_id(0)

    @pl.when(meta_ref[i, 1] == 1)
    def _():
        xb = xs_ref[...]
        wg = w1_ref[0, pl.ds(0, DF), :]
        wu = w1_ref[0, pl.ds(DF, DF), :]
        g = lax.dot_general(xb, wg, (((1,), (1,)), ((), ())),
                            preferred_element_type=jnp.float32)
        u = lax.dot_general(xb, wu, (((1,), (1,)), ((), ())),
                            preferred_element_type=jnp.float32)
        a = g * (1.0 / (1.0 + jnp.exp(-g))) * u
        ys_ref[...] = lax.dot_general(a, w2_ref[0], (((1,), (1,)), ((), ())),
                                      preferred_element_type=jnp.float32)


def _gmm(meta, xs, w1, w2):
    grid_spec = pltpu.PrefetchScalarGridSpec(
        num_scalar_prefetch=1,
        grid=(NB,),
        in_specs=[
            pl.BlockSpec((BM, D), lambda i, meta: (i, 0)),
            pl.BlockSpec((1, 2 * DF, D), lambda i, meta: (meta[i, 0], 0, 0)),
            pl.BlockSpec((1, D, DF), lambda i, meta: (meta[i, 0], 0, 0)),
        ],
        out_specs=pl.BlockSpec((BM, D), lambda i, meta: (i, 0)),
    )
    return pl.pallas_call(
        _gmm_body,
        grid_spec=grid_spec,
        out_shape=jax.ShapeDtypeStruct((TPAD, D), jnp.float32),
    )(meta, xs, w1, w2)


def _sc_mesh():
    info = plsc.get_sparse_core_info()
    return (plsc.VectorSubcoreMesh(core_axis_name="c", subcore_axis_name="s"),
            info.num_cores, info.num_subcores)


def _dispatch(x, pos_flat):
    mesh, nc, ns = _sc_mesh()
    nw = nc * ns
    ppw = (M * TOPK) // nw
    half = ppw // 2

    def body(x_hbm, posf_hbm, xs_hbm, idx_v, tok_v, rows_v):
        wid = lax.axis_index("s") * nc + lax.axis_index("c")
        pair_base = wid * ppw
        for ch in range(2):
            pltpu.sync_copy(posf_hbm.at[pl.ds(pair_base + ch * half, half)],
                            idx_v.at[ch])
        lane = lax.broadcasted_iota(jnp.int32, (16,), 0)
        for ch in range(2):
            for k in range(half // 16):
                j = pair_base + ch * half + k * 16
                tok_v[ch, pl.ds(k * 16, 16)] = lax.shift_right_logical(
                    j + lane, 1)
        for ch in range(2):
            pltpu.sync_copy(x_hbm.at[tok_v.at[ch]], rows_v)
            pltpu.sync_copy(rows_v, xs_hbm.at[idx_v.at[ch]])

    f = pl.kernel(
        body,
        out_type=jax.ShapeDtypeStruct((TPAD, D), jnp.float32),
        mesh=mesh,
        scratch_types=[
            pltpu.VMEM((2, half), jnp.int32),
            pltpu.VMEM((2, half), jnp.int32),
            pltpu.VMEM((half, D), jnp.float32),
        ],
    )
    return f(x, pos_flat)


def _combine(ys, pos_flat, wts_flat):
    mesh, nc, ns = _sc_mesh()
    nw = nc * ns
    tpw = M // nw
    ppw = tpw * TOPK
    sub = 16
    nsub = tpw // sub

    def body(ys_hbm, posf_hbm, wtsf_hbm, out_hbm, idx_v, wf_v, buf_v, out_v):
        wid = lax.axis_index("s") * nc + lax.axis_index("c")
        tok_base = wid * tpw
        pair_base = tok_base * TOPK
        pltpu.sync_copy(wtsf_hbm.at[pl.ds(pair_base, ppw)],
                        wf_v.at[pl.ds(0, ppw)])
        for ch in range(nsub):
            pltpu.sync_copy(
                posf_hbm.at[pl.ds(pair_base + ch * sub * TOPK, sub * TOPK)],
                idx_v.at[ch])
        for ch in range(nsub):
            pltpu.sync_copy(ys_hbm.at[idx_v.at[ch]], buf_v)

            @pl.loop(0, sub)
            def _(i):
                wv = wf_v[pl.ds(ch * sub * TOPK + 2 * i, 16)]
                w0 = wv[0]
                w1 = wv[1]
                for d in range(D // 16):
                    sl = pl.ds(d * 16, 16)
                    out_v[i, sl] = buf_v[2 * i, sl] * w0 + buf_v[2 * i + 1, sl] * w1

            pltpu.sync_copy(out_v, out_hbm.at[pl.ds(tok_base + ch * sub, sub)])

    f = pl.kernel(
        body,
        out_type=jax.ShapeDtypeStruct((M, D), jnp.float32),
        mesh=mesh,
        scratch_types=[
            pltpu.VMEM((nsub, sub * TOPK), jnp.int32),
            pltpu.VMEM((ppw + 16,), jnp.float32),
            pltpu.VMEM((sub * TOPK, D), jnp.float32),
            pltpu.VMEM((sub, D), jnp.float32),
        ],
    )
    return f(ys, pos_flat, wts_flat)


def kernel(x, router_logits, w1, w2):
    pos, wts, meta = _route(router_logits)
    pos_flat = pos.reshape(M * TOPK)
    wts_flat = wts.reshape(M * TOPK)
    xs = _dispatch(x, pos_flat)
    ys = _gmm(meta, xs, w1, w2)
    return _combine(ys, pos_flat, wts_flat)
